# baseline (device time: 12302 ns/iter reference)
import functools

import jax
import jax.numpy as jnp
from jax import lax
from jax.experimental import pallas as pl
from jax.experimental.pallas import tpu as pltpu

M = 512
H = M // 2
K = 4
C = H // K


def kernel(x):
    m_per, n = x.shape
    assert m_per == M and n == 2 * M

    def body(x_ref, out_ref, send_buf, sx_send, sx_recv, sy_send, sy_recv):
        my_y = lax.axis_index("y")
        yn = 1 - my_y

        def run(px):
            pp = 1 - px
            my_x = lax.axis_index("x")

            barrier_sem = pltpu.get_barrier_semaphore()
            for dev in ((pp, my_y), (px, yn)):
                pl.semaphore_signal(
                    barrier_sem, inc=1,
                    device_id=dev, device_id_type=pl.DeviceIdType.MESH,
                )
            pl.semaphore_wait(barrier_sem, 2)

            def x_send(k):
                return pltpu.make_async_remote_copy(
                    src_ref=send_buf.at[pl.ds(k * C, C), :],
                    dst_ref=out_ref.at[pl.ds(px * M + my_y * H + k * C, C), :],
                    send_sem=sx_send.at[k],
                    recv_sem=sx_recv.at[k],
                    device_id=(pp, my_y),
                    device_id_type=pl.DeviceIdType.MESH,
                )

            def x_recv(k):
                return pltpu.make_async_remote_copy(
                    src_ref=send_buf.at[pl.ds(k * C, C), :],
                    dst_ref=out_ref.at[pl.ds(pp * M + my_y * H + k * C, C), :],
                    send_sem=sx_send.at[k],
                    recv_sem=sx_recv.at[k],
                    device_id=(pp, my_y),
                    device_id_type=pl.DeviceIdType.MESH,
                )

            def y_fwd(k):
                return pltpu.make_async_remote_copy(
                    src_ref=out_ref.at[pl.ds(pp * M + my_y * H + k * C, C), :],
                    dst_ref=out_ref.at[pl.ds(pp * M + my_y * H + k * C, C), :],
                    send_sem=sy_send.at[k],
                    recv_sem=sy_recv.at[k],
                    device_id=(px, yn),
                    device_id_type=pl.DeviceIdType.MESH,
                )

            def y_recv(k):
                return pltpu.make_async_remote_copy(
                    src_ref=out_ref.at[pl.ds(pp * M + my_y * H + k * C, C), :],
                    dst_ref=out_ref.at[pl.ds(pp * M + yn * H + k * C, C), :],
                    send_sem=sy_send.at[k],
                    recv_sem=sy_recv.at[k],
                    device_id=(px, yn),
                    device_id_type=pl.DeviceIdType.MESH,
                )

            for k in range(K):
                send_buf[pl.ds(k * C, C), :] = (
                    x_ref[pl.ds(my_y * H + k * C, C), pp * M:(pp + 1) * M]
                    .astype(jnp.bfloat16)
                )
                x_send(k).start()

            out_ref[pl.ds(px * M, M), :] = (
                x_ref[:, px * M:(px + 1) * M].astype(jnp.bfloat16)
            )

            for k in range(K):
                x_recv(k).wait_recv()
                y_fwd(k).start()

            for k in range(K):
                y_recv(k).wait_recv()
            for k in range(K):
                x_send(k).wait_send()
                y_fwd(k).wait_send()

        pl.when(lax.axis_index("x") == 0)(functools.partial(run, 0))
        pl.when(lax.axis_index("x") == 1)(functools.partial(run, 1))

    return pl.pallas_call(
        body,
        out_shape=jax.ShapeDtypeStruct((2 * M, M), jnp.bfloat16),
        in_specs=[pl.BlockSpec(memory_space=pltpu.VMEM)],
        out_specs=pl.BlockSpec(memory_space=pltpu.VMEM),
        scratch_shapes=[
            pltpu.VMEM((H, M), jnp.bfloat16),
            pltpu.SemaphoreType.DMA((K,)),
            pltpu.SemaphoreType.DMA((K,)),
            pltpu.SemaphoreType.DMA((K,)),
            pltpu.SemaphoreType.DMA((K,)),
        ],
        compiler_params=pltpu.CompilerParams(collective_id=0),
    )(x)


# device time: 12065 ns/iter; 1.0196x vs baseline; 1.0196x over previous
import functools

import jax
import jax.numpy as jnp
from jax import lax
from jax.experimental import pallas as pl
from jax.experimental.pallas import tpu as pltpu

M = 512
H = M // 2
K = 8
C = H // K


def kernel(x):
    m_per, n = x.shape
    assert m_per == M and n == 2 * M

    def body(x_ref, out_ref, send_buf, sx_send, sx_recv, sy_send, sy_recv):
        my_y = lax.axis_index("y")
        yn = 1 - my_y

        def run(px):
            pp = 1 - px
            my_x = lax.axis_index("x")

            barrier_sem = pltpu.get_barrier_semaphore()
            for dev in ((pp, my_y), (px, yn)):
                pl.semaphore_signal(
                    barrier_sem, inc=1,
                    device_id=dev, device_id_type=pl.DeviceIdType.MESH,
                )

            def x_send(k):
                return pltpu.make_async_remote_copy(
                    src_ref=send_buf.at[pl.ds(k * C, C), :],
                    dst_ref=out_ref.at[pl.ds(px * M + my_y * H + k * C, C), :],
                    send_sem=sx_send.at[k],
                    recv_sem=sx_recv.at[k],
                    device_id=(pp, my_y),
                    device_id_type=pl.DeviceIdType.MESH,
                )

            def x_recv(k):
                return pltpu.make_async_remote_copy(
                    src_ref=send_buf.at[pl.ds(k * C, C), :],
                    dst_ref=out_ref.at[pl.ds(pp * M + my_y * H + k * C, C), :],
                    send_sem=sx_send.at[k],
                    recv_sem=sx_recv.at[k],
                    device_id=(pp, my_y),
                    device_id_type=pl.DeviceIdType.MESH,
                )

            def y_fwd(k):
                return pltpu.make_async_remote_copy(
                    src_ref=out_ref.at[pl.ds(pp * M + my_y * H + k * C, C), :],
                    dst_ref=out_ref.at[pl.ds(pp * M + my_y * H + k * C, C), :],
                    send_sem=sy_send.at[k],
                    recv_sem=sy_recv.at[k],
                    device_id=(px, yn),
                    device_id_type=pl.DeviceIdType.MESH,
                )

            def y_recv(k):
                return pltpu.make_async_remote_copy(
                    src_ref=out_ref.at[pl.ds(pp * M + my_y * H + k * C, C), :],
                    dst_ref=out_ref.at[pl.ds(pp * M + yn * H + k * C, C), :],
                    send_sem=sy_send.at[k],
                    recv_sem=sy_recv.at[k],
                    device_id=(px, yn),
                    device_id_type=pl.DeviceIdType.MESH,
                )

            send_buf[...] = (
                x_ref[pl.ds(my_y * H, H), pp * M:(pp + 1) * M]
                .astype(jnp.bfloat16)
            )
            out_ref[pl.ds(px * M, M), :] = (
                x_ref[:, px * M:(px + 1) * M].astype(jnp.bfloat16)
            )

            pl.semaphore_wait(barrier_sem, 2)

            for k in range(K):
                x_send(k).start()

            for k in range(K):
                x_recv(k).wait_recv()
                y_fwd(k).start()

            for k in range(K):
                y_recv(k).wait_recv()
            for k in range(K):
                x_send(k).wait_send()
                y_fwd(k).wait_send()

        pl.when(lax.axis_index("x") == 0)(functools.partial(run, 0))
        pl.when(lax.axis_index("x") == 1)(functools.partial(run, 1))

    return pl.pallas_call(
        body,
        out_shape=jax.ShapeDtypeStruct((2 * M, M), jnp.bfloat16),
        in_specs=[pl.BlockSpec(memory_space=pltpu.VMEM)],
        out_specs=pl.BlockSpec(memory_space=pltpu.VMEM),
        scratch_shapes=[
            pltpu.VMEM((H, M), jnp.bfloat16),
            pltpu.SemaphoreType.DMA((K,)),
            pltpu.SemaphoreType.DMA((K,)),
            pltpu.SemaphoreType.DMA((K,)),
            pltpu.SemaphoreType.DMA((K,)),
        ],
        compiler_params=pltpu.CompilerParams(collective_id=0),
    )(x)


# device time: 11355 ns/iter; 1.0834x vs baseline; 1.0625x over previous
import functools

import jax
import jax.numpy as jnp
from jax import lax
from jax.experimental import pallas as pl
from jax.experimental.pallas import tpu as pltpu

M = 512
S = 320
F = M - S
C = 32
K = S // C
J = F // C


def kernel(x):
    m_per, n = x.shape
    assert m_per == M and n == 2 * M

    def body(x_ref, out_ref, send_buf, sx_send, sx_recv, sy_send, sy_recv):
        my_y = lax.axis_index("y")
        yn = 1 - my_y

        def row(j, py):
            return jnp.where(py == 0, j * C, M - (j + 1) * C)

        def run(px):
            pp = 1 - px
            my_row = functools.partial(row, py=my_y)
            nb_row = functools.partial(row, py=yn)

            barrier_sem = pltpu.get_barrier_semaphore()
            for dev in ((pp, my_y), (px, yn)):
                pl.semaphore_signal(
                    barrier_sem, inc=1,
                    device_id=dev, device_id_type=pl.DeviceIdType.MESH,
                )

            def x_send(j):
                r = my_row(j)
                return pltpu.make_async_remote_copy(
                    src_ref=send_buf.at[pl.ds(r, C), :],
                    dst_ref=out_ref.at[pl.ds(px * M + r, C), :],
                    send_sem=sx_send.at[j],
                    recv_sem=sx_recv.at[j],
                    device_id=(pp, my_y),
                    device_id_type=pl.DeviceIdType.MESH,
                )

            def x_recv(j):
                r = my_row(j)
                return pltpu.make_async_remote_copy(
                    src_ref=send_buf.at[pl.ds(r, C), :],
                    dst_ref=out_ref.at[pl.ds(pp * M + r, C), :],
                    send_sem=sx_send.at[j],
                    recv_sem=sx_recv.at[j],
                    device_id=(pp, my_y),
                    device_id_type=pl.DeviceIdType.MESH,
                )

            def y_fwd(j):
                r = my_row(j)
                return pltpu.make_async_remote_copy(
                    src_ref=out_ref.at[pl.ds(pp * M + r, C), :],
                    dst_ref=out_ref.at[pl.ds(pp * M + r, C), :],
                    send_sem=sy_send.at[j],
                    recv_sem=sy_recv.at[j],
                    device_id=(px, yn),
                    device_id_type=pl.DeviceIdType.MESH,
                )

            def y_recv(j):
                r = nb_row(j)
                return pltpu.make_async_remote_copy(
                    src_ref=out_ref.at[pl.ds(pp * M + r, C), :],
                    dst_ref=out_ref.at[pl.ds(pp * M + r, C), :],
                    send_sem=sy_send.at[j],
                    recv_sem=sy_recv.at[j],
                    device_id=(px, yn),
                    device_id_type=pl.DeviceIdType.MESH,
                )

            send_buf[...] = x_ref[:, pp * M:(pp + 1) * M].astype(jnp.bfloat16)
            out_ref[pl.ds(px * M, M), :] = (
                x_ref[:, px * M:(px + 1) * M].astype(jnp.bfloat16)
            )

            pl.semaphore_wait(barrier_sem, 2)

            for j in range(K):
                x_send(j).start()

            for j in range(J):
                x_recv(j).wait_recv()
                y_fwd(j).start()
            for j in range(J, K):
                x_recv(j).wait_recv()

            for j in range(J):
                y_recv(j).wait_recv()
            for j in range(K):
                x_send(j).wait_send()
            for j in range(J):
                y_fwd(j).wait_send()

        pl.when(lax.axis_index("x") == 0)(functools.partial(run, 0))
        pl.when(lax.axis_index("x") == 1)(functools.partial(run, 1))

    return pl.pallas_call(
        body,
        out_shape=jax.ShapeDtypeStruct((2 * M, M), jnp.bfloat16),
        in_specs=[pl.BlockSpec(memory_space=pltpu.VMEM)],
        out_specs=pl.BlockSpec(memory_space=pltpu.VMEM),
        scratch_shapes=[
            pltpu.VMEM((M, M), jnp.bfloat16),
            pltpu.SemaphoreType.DMA((K,)),
            pltpu.SemaphoreType.DMA((K,)),
            pltpu.SemaphoreType.DMA((J,)),
            pltpu.SemaphoreType.DMA((J,)),
        ],
        compiler_params=pltpu.CompilerParams(collective_id=0),
    )(x)


# device time: 11354 ns/iter; 1.0835x vs baseline; 1.0001x over previous
import functools

import jax
import jax.numpy as jnp
from jax import lax
from jax.experimental import pallas as pl
from jax.experimental.pallas import tpu as pltpu

M = 512
S = 320
F = M - S
C = 32
K = S // C
J = F // C


def kernel(x):
    m_per, n = x.shape
    assert m_per == M and n == 2 * M

    def body(x_ref, out_ref, send_buf, sx_send, sx_recv, sy_send, sy_recv):
        my_y = lax.axis_index("y")
        yn = 1 - my_y

        def row(j, py):
            return jnp.where(py == 0, j * C, M - (j + 1) * C)

        def run(px):
            pp = 1 - px
            my_row = functools.partial(row, py=my_y)
            nb_row = functools.partial(row, py=yn)

            barrier_sem = pltpu.get_barrier_semaphore()
            for dev in ((pp, my_y), (px, yn)):
                pl.semaphore_signal(
                    barrier_sem, inc=1,
                    device_id=dev, device_id_type=pl.DeviceIdType.MESH,
                )

            def x_send(j):
                r = my_row(j)
                return pltpu.make_async_remote_copy(
                    src_ref=send_buf.at[pl.ds(r, C), :],
                    dst_ref=out_ref.at[pl.ds(px * M + r, C), :],
                    send_sem=sx_send.at[j],
                    recv_sem=sx_recv.at[j],
                    device_id=(pp, my_y),
                    device_id_type=pl.DeviceIdType.MESH,
                )

            def x_recv(j):
                r = my_row(j)
                return pltpu.make_async_remote_copy(
                    src_ref=send_buf.at[pl.ds(r, C), :],
                    dst_ref=out_ref.at[pl.ds(pp * M + r, C), :],
                    send_sem=sx_send.at[j],
                    recv_sem=sx_recv.at[j],
                    device_id=(pp, my_y),
                    device_id_type=pl.DeviceIdType.MESH,
                )

            def y_fwd(j):
                r = my_row(j)
                return pltpu.make_async_remote_copy(
                    src_ref=out_ref.at[pl.ds(pp * M + r, C), :],
                    dst_ref=out_ref.at[pl.ds(pp * M + r, C), :],
                    send_sem=sy_send.at[j],
                    recv_sem=sy_recv.at[j],
                    device_id=(px, yn),
                    device_id_type=pl.DeviceIdType.MESH,
                )

            def y_recv(j):
                r = nb_row(j)
                return pltpu.make_async_remote_copy(
                    src_ref=out_ref.at[pl.ds(pp * M + r, C), :],
                    dst_ref=out_ref.at[pl.ds(pp * M + r, C), :],
                    send_sem=sy_send.at[j],
                    recv_sem=sy_recv.at[j],
                    device_id=(px, yn),
                    device_id_type=pl.DeviceIdType.MESH,
                )

            pl.semaphore_wait(barrier_sem, 2)

            for j in range(K):
                r = my_row(j)
                send_buf[pl.ds(r, C), :] = (
                    x_ref[pl.ds(r, C), pp * M:(pp + 1) * M]
                    .astype(jnp.bfloat16)
                )
                x_send(j).start()

            out_ref[pl.ds(px * M, M), :] = (
                x_ref[:, px * M:(px + 1) * M].astype(jnp.bfloat16)
            )

            for j in range(J):
                x_recv(j).wait_recv()
                y_fwd(j).start()
            for j in range(J, K):
                x_recv(j).wait_recv()

            for j in range(J):
                y_recv(j).wait_recv()
            for j in range(K):
                x_send(j).wait_send()
            for j in range(J):
                y_fwd(j).wait_send()

        pl.when(lax.axis_index("x") == 0)(functools.partial(run, 0))
        pl.when(lax.axis_index("x") == 1)(functools.partial(run, 1))

    return pl.pallas_call(
        body,
        out_shape=jax.ShapeDtypeStruct((2 * M, M), jnp.bfloat16),
        in_specs=[pl.BlockSpec(memory_space=pltpu.VMEM)],
        out_specs=pl.BlockSpec(memory_space=pltpu.VMEM),
        scratch_shapes=[
            pltpu.VMEM((M, M), jnp.bfloat16),
            pltpu.SemaphoreType.DMA((K,)),
            pltpu.SemaphoreType.DMA((K,)),
            pltpu.SemaphoreType.DMA((J,)),
            pltpu.SemaphoreType.DMA((J,)),
        ],
        compiler_params=pltpu.CompilerParams(collective_id=0),
    )(x)
